# baseline (speedup 1.0000x reference)
import functools

import jax
import jax.numpy as jnp
from jax import lax
from jax.experimental import pallas as pl
from jax.experimental.pallas import tpu as pltpu
from jax.experimental.pallas import tpu_sc as plsc

NUM_CORES = 2
NUM_SUBCORES = 16
NUM_TILES = NUM_CORES * NUM_SUBCORES
ROWS_PER_CHUNK = 8
NBUF = 4


@functools.partial(jax.jit, static_argnames=("batch", "seq", "dim"))
def _sc_embed(embed_weight, input_ids, batch, seq, dim):
    rows_per_tile = (batch * seq) // NUM_TILES
    num_chunks = rows_per_tile // ROWS_PER_CHUNK
    tiles_per_batch_row = seq // rows_per_tile
    mesh = plsc.VectorSubcoreMesh(core_axis_name="c", subcore_axis_name="s")

    @functools.partial(
        pl.kernel,
        out_type=jax.ShapeDtypeStruct((batch, seq, dim), jnp.float32),
        mesh=mesh,
        scratch_types=[
            pltpu.VMEM((rows_per_tile,), jnp.int32),
        ]
        + [pltpu.VMEM((ROWS_PER_CHUNK, dim), jnp.float32)] * NBUF
        + [pltpu.SemaphoreType.DMA] * (2 * NBUF),
    )
    def k(table_hbm, idx_hbm, out_hbm, idx_v, *bufs_and_sems):
        bufs = bufs_and_sems[:NBUF]
        gsems = bufs_and_sems[NBUF : 2 * NBUF]
        ssems = bufs_and_sems[2 * NBUF :]
        wid = lax.axis_index("s") * NUM_CORES + lax.axis_index("c")
        b = wid // tiles_per_batch_row
        off = (wid % tiles_per_batch_row) * rows_per_tile
        pltpu.sync_copy(idx_hbm.at[b, pl.ds(off, rows_per_tile)], idx_v)
        R = ROWS_PER_CHUNK

        def fire_gather(c, i):
            pltpu.async_copy(
                table_hbm.at[idx_v.at[pl.ds(c * R, R)]], bufs[i], gsems[i]
            )

        def wait_gather(c, i):
            pltpu.make_async_copy(
                table_hbm.at[idx_v.at[pl.ds(c * R, R)]], bufs[i], gsems[i]
            ).wait()

        def fire_store(c, i):
            pltpu.async_copy(
                bufs[i], out_hbm.at[b, pl.ds(off + c * R, R)], ssems[i]
            )

        def wait_store(c, i):
            pltpu.make_async_copy(
                bufs[i], out_hbm.at[b, pl.ds(off + c * R, R)], ssems[i]
            ).wait()

        fire_gather(0, 0)
        fire_gather(1, 1)
        wait_gather(0, 0)
        fire_store(0, 0)
        fire_gather(2, 2)
        wait_gather(1, 1)
        fire_store(1, 1)
        fire_gather(3, 3)

        @pl.loop(2, num_chunks - 2 - ((num_chunks - 4) % NBUF), step=NBUF)
        def _(j):
            for i in range(NBUF):
                c = j + i
                bi = (2 + i) % NBUF
                wait_gather(c, bi)
                fire_store(c, bi)
                wait_store(c - 2, (bi - 2) % NBUF)
                fire_gather(c + 2, (bi + 2) % NBUF)

        rem_start = num_chunks - 2 - ((num_chunks - 4) % NBUF)
        for c in range(rem_start, num_chunks - 2):
            bi = c % NBUF
            wait_gather(c, bi)
            fire_store(c, bi)
            wait_store(c - 2, (bi - 2) % NBUF)
            fire_gather(c + 2, (bi + 2) % NBUF)

        for c in range(num_chunks - 2, num_chunks):
            bi = c % NBUF
            wait_gather(c, bi)
            fire_store(c, bi)
            wait_store(c - 2, (bi - 2) % NBUF)
        wait_store(num_chunks - 2, (num_chunks - 2) % NBUF)
        wait_store(num_chunks - 1, (num_chunks - 1) % NBUF)

    return k(embed_weight, input_ids)


def kernel(input_ids, embed_weight):
    batch, seq = input_ids.shape
    dim = embed_weight.shape[1]
    return _sc_embed(embed_weight, input_ids, batch, seq, dim)
